# baseline (speedup 1.0000x reference)
import jax
import jax.numpy as jnp
from jax.experimental import pallas as pl
from jax.experimental.pallas import tpu as pltpu
from jax.experimental.pallas import tpu_sc as plsc

_L = 512
_D = 128
_BI = 32
_NG = 2 * _L + 1024
_WIN = 128


def _sc_gather(tab, idx_all):
    idx2d = idx_all.reshape(1, _NG)

    @pl.kernel(
        out_type=jax.ShapeDtypeStruct((_NG, _D), jnp.float32),
        mesh=plsc.VectorSubcoreMesh(core_axis_name="core",
                                    subcore_axis_name="subcore"),
    )
    def gather_kernel(tab_hbm, idx_hbm, g_hbm):
        def body(i_vmem, o_vmem):
            pltpu.sync_copy(tab_hbm.at[i_vmem.at[0]], o_vmem)

        pltpu.emit_pipeline(
            body,
            grid=(_NG // _WIN,),
            in_specs=[pl.BlockSpec((1, _WIN), index_map=lambda i: (0, i))],
            out_specs=[pl.BlockSpec((_WIN, _D), index_map=lambda i: (i, 0))],
            core_axis_name=("core", "subcore"),
            dimension_semantics=(pltpu.PARALLEL,),
        )(idx_hbm, g_hbm)

    return gather_kernel(tab, idx2d)


def _edge_body(g_ref, x_ref, o_ref):
    i0 = pl.program_id(0) * _BI
    pj = g_ref[_L:2 * _L, :]
    for r in range(_BI):
        pi = g_ref[pl.ds(i0 + r, 1), :]
        rel = g_ref[pl.ds(2 * _L + _L - 1 - (i0 + r), _L), :]
        o_ref[r] = x_ref[r] + pi + pj + rel


def kernel(fasta_sequence, out, W_i, W_j, W_rel):
    seq = fasta_sequence.reshape(_L).astype(jnp.int32)
    n_i = W_i.shape[0]
    n_rel = W_rel.shape[0]
    one_side = n_rel // 2

    k = jnp.arange(1024, dtype=jnp.int32)
    rel_idx = jnp.clip(k - (_L - 1), -one_side, one_side) + one_side
    idx_all = jnp.concatenate([seq, seq + n_i, rel_idx + n_i + W_j.shape[0]])

    tab = jnp.concatenate([W_i, W_j, W_rel], axis=0)
    tab = jnp.pad(tab, ((0, 128 - tab.shape[0]), (0, 0)))

    g = _sc_gather(tab, idx_all)

    x = out.reshape(_L, _L, _D)
    res = pl.pallas_call(
        _edge_body,
        grid=(_L // _BI,),
        in_specs=[
            pl.BlockSpec((_NG, _D), lambda i: (0, 0)),
            pl.BlockSpec((_BI, _L, _D), lambda i: (i, 0, 0)),
        ],
        out_specs=pl.BlockSpec((_BI, _L, _D), lambda i: (i, 0, 0)),
        out_shape=jax.ShapeDtypeStruct((_L, _L, _D), jnp.float32),
    )(g, x)
    return res.reshape(out.shape)


# speedup vs baseline: 1.1847x; 1.1847x over previous
import jax
import jax.numpy as jnp
from jax.experimental import pallas as pl
from jax.experimental.pallas import tpu as pltpu
from jax.experimental.pallas import tpu_sc as plsc

_L = 512
_D = 128
_BI = 32
_NG = 2 * _L
_WIN = 128


def _sc_gather(tab, idx_all):
    idx2d = idx_all.reshape(1, _NG)

    @pl.kernel(
        out_type=jax.ShapeDtypeStruct((_NG, _D), jnp.float32),
        mesh=plsc.VectorSubcoreMesh(core_axis_name="core",
                                    subcore_axis_name="subcore"),
    )
    def gather_kernel(tab_hbm, idx_hbm, g_hbm):
        def body(i_vmem, o_vmem):
            pltpu.sync_copy(tab_hbm.at[i_vmem.at[0]], o_vmem)

        pltpu.emit_pipeline(
            body,
            grid=(_NG // _WIN,),
            in_specs=[pl.BlockSpec((1, _WIN), index_map=lambda i: (0, i))],
            out_specs=[pl.BlockSpec((_WIN, _D), index_map=lambda i: (i, 0))],
            core_axis_name=("core", "subcore"),
            dimension_semantics=(pltpu.PARALLEL,),
        )(idx_hbm, g_hbm)

    return gather_kernel(tab, idx2d)


def _edge_body(wrel_ref, g_ref, x_ref, o_ref, r3_ref):
    @pl.when(pl.program_id(0) == 0)
    def _build_r3():
        k = jax.lax.broadcasted_iota(jnp.int32, (1024, 1), 0)
        ridx = jnp.clip(k - (_L - 1), -32, 32) + 32
        onehot = (jax.lax.broadcasted_iota(jnp.int32, (1024, 128), 1)
                  == ridx).astype(jnp.float32)
        r3_ref[...] = jax.lax.dot_general(
            onehot, wrel_ref[...], (((1,), (0,)), ((), ())),
            preferred_element_type=jnp.float32)

    i0 = pl.program_id(0) * _BI
    pj = g_ref[_L:2 * _L, :]
    for r in range(_BI):
        pi = g_ref[pl.ds(i0 + r, 1), :]
        rel = r3_ref[pl.ds(_L - 1 - (i0 + r), _L), :]
        o_ref[r] = x_ref[r] + pi + pj + rel


def kernel(fasta_sequence, out, W_i, W_j, W_rel):
    seq = fasta_sequence.reshape(_L).astype(jnp.int32)
    n_i = W_i.shape[0]

    idx_all = jnp.concatenate([seq, seq + n_i])
    tab = jnp.concatenate([W_i, W_j], axis=0)

    g = _sc_gather(tab, idx_all)

    wrel_pad = jnp.pad(W_rel, ((0, 128 - W_rel.shape[0]), (0, 0)))

    x = out.reshape(_L, _L, _D)
    res = pl.pallas_call(
        _edge_body,
        grid=(_L // _BI,),
        in_specs=[
            pl.BlockSpec((128, _D), lambda i: (0, 0)),
            pl.BlockSpec((_NG, _D), lambda i: (0, 0)),
            pl.BlockSpec((_BI, _L, _D), lambda i: (i, 0, 0)),
        ],
        out_specs=pl.BlockSpec((_BI, _L, _D), lambda i: (i, 0, 0)),
        out_shape=jax.ShapeDtypeStruct((_L, _L, _D), jnp.float32),
        scratch_shapes=[pltpu.VMEM((1024, _D), jnp.float32)],
    )(wrel_pad, g, x)
    return res.reshape(out.shape)
